# Initial kernel scaffold; baseline (speedup 1.0000x reference)
import jax
import jax.numpy as jnp
from jax.experimental import pallas as pl


def kernel(hidden, router_w, expert_bias, Wg, Wu, Wd, SWg, SWu, SWd):
    raise NotImplementedError("write your pallas kernel here")


# speedup vs baseline: 1.1079x; 1.1079x over previous
import functools

import jax
import jax.numpy as jnp
from jax.experimental import pallas as pl
from jax.experimental.pallas import tpu as pltpu

E = 8
TOPK = 2
D = 2048
F = 1024
T = 2048
ALPHA = 7.0
LANES = 128


def _router_body(x_ref, rwt_ref, bias_ref, probs_ref):
    x = x_ref[...]
    logits = jnp.dot(x, rwt_ref[...], preferred_element_type=jnp.float32)
    scores = jnp.sqrt(jnp.logaddexp(logits, 0.0))
    li = jax.lax.broadcasted_iota(jnp.int32, (T, LANES), 1)
    valid = li < E
    neg = jnp.float32(-jnp.inf)
    sel = jnp.where(valid, scores + bias_ref[...], neg)
    m1 = jnp.max(sel, axis=1, keepdims=True)
    i1 = jnp.min(jnp.where(sel == m1, li, LANES), axis=1, keepdims=True)
    sel2 = jnp.where(li == i1, neg, sel)
    m2 = jnp.max(sel2, axis=1, keepdims=True)
    i2 = jnp.min(jnp.where(sel2 == m2, li, LANES), axis=1, keepdims=True)
    s1 = jnp.sum(jnp.where(li == i1, scores, 0.0), axis=1, keepdims=True)
    s2 = jnp.sum(jnp.where(li == i2, scores, 0.0), axis=1, keepdims=True)
    denom = s1 + s2 + 1e-20
    w1 = s1 / denom
    w2 = s2 / denom
    probs_ref[...] = jnp.where(li == i1, w1, 0.0) + jnp.where(li == i2, w2, 0.0)


def _route_pallas(flat, rw_pad, bias_pad):
    return pl.pallas_call(
        _router_body,
        out_shape=jax.ShapeDtypeStruct((T, LANES), jnp.float32),
    )(flat, rw_pad, bias_pad)


BT = 512
BF = 512


def _ffn_body(x_ref, wg_ref, wu_ref, wd_ref, p_ref, o_ref):
    e = pl.program_id(1)
    f = pl.program_id(2)

    @pl.when((e == 0) & (f == 0))
    def _():
        o_ref[...] = jnp.zeros_like(o_ref)

    x = x_ref[...]
    g = jnp.dot(x, wg_ref[0], preferred_element_type=jnp.float32)
    u = jnp.dot(x, wu_ref[0], preferred_element_type=jnp.float32)
    g = jnp.minimum(g, ALPHA)
    u = jnp.clip(u, -ALPHA, ALPHA)
    act = g * jax.nn.sigmoid(g) * u
    y = jnp.dot(act, wd_ref[0], preferred_element_type=jnp.float32)
    o_ref[...] += y * p_ref[0]


def _ffn_dense(flat, Wg9, Wu9, Wd9, probs9):
    grid = (T // BT, E + 1, F // BF)
    return pl.pallas_call(
        _ffn_body,
        grid=grid,
        in_specs=[
            pl.BlockSpec((BT, D), lambda t, e, f: (t, 0)),
            pl.BlockSpec((1, D, BF), lambda t, e, f: (e, 0, f)),
            pl.BlockSpec((1, D, BF), lambda t, e, f: (e, 0, f)),
            pl.BlockSpec((1, BF, D), lambda t, e, f: (e, f, 0)),
            pl.BlockSpec((1, BT, 1), lambda t, e, f: (e, t, 0)),
        ],
        out_specs=pl.BlockSpec((BT, D), lambda t, e, f: (t, 0)),
        out_shape=jax.ShapeDtypeStruct((T, D), jnp.float32),
    )(flat, Wg9, Wu9, Wd9, probs9)


@jax.jit
def kernel(hidden, router_w, expert_bias, Wg, Wu, Wd, SWg, SWu, SWd):
    B, S, Dm = hidden.shape
    flat = hidden.reshape(T, D)

    rw_pad = jnp.zeros((D, LANES), jnp.float32).at[:, :E].set(router_w.T)
    bias_pad = jnp.zeros((1, LANES), jnp.float32).at[:, :E].set(
        expert_bias[None, :])
    probs = _route_pallas(flat, rw_pad, bias_pad)

    probs9 = jnp.concatenate(
        [probs[:, :E], jnp.ones((T, 1), jnp.float32)], axis=1)
    probs9 = probs9.T.reshape(E + 1, T, 1)

    Wg9 = jnp.concatenate([Wg, SWg[None]], axis=0)
    Wu9 = jnp.concatenate([Wu, SWu[None]], axis=0)
    Wd9 = jnp.concatenate([Wd, SWd[None]], axis=0)

    out = _ffn_dense(flat, Wg9, Wu9, Wd9, probs9)
    return out.reshape(B, S, Dm)


# speedup vs baseline: 1.1632x; 1.0499x over previous
import functools

import jax
import jax.numpy as jnp
from jax.experimental import pallas as pl
from jax.experimental.pallas import tpu as pltpu

E = 8
TOPK = 2
D = 2048
F = 1024
T = 2048
ALPHA = 7.0
LANES = 128


def _router_body(x_ref, rwt_ref, bias_ref, probs_ref):
    x = x_ref[...]
    logits = jnp.dot(x, rwt_ref[...], preferred_element_type=jnp.float32)
    scores = jnp.sqrt(jnp.logaddexp(logits, 0.0))
    li = jax.lax.broadcasted_iota(jnp.int32, (T, LANES), 1)
    valid = li < E
    neg = jnp.float32(-jnp.inf)
    sel = jnp.where(valid, scores + bias_ref[...], neg)
    m1 = jnp.max(sel, axis=1, keepdims=True)
    i1 = jnp.min(jnp.where(sel == m1, li, LANES), axis=1, keepdims=True)
    sel2 = jnp.where(li == i1, neg, sel)
    m2 = jnp.max(sel2, axis=1, keepdims=True)
    i2 = jnp.min(jnp.where(sel2 == m2, li, LANES), axis=1, keepdims=True)
    s1 = jnp.sum(jnp.where(li == i1, scores, 0.0), axis=1, keepdims=True)
    s2 = jnp.sum(jnp.where(li == i2, scores, 0.0), axis=1, keepdims=True)
    denom = s1 + s2 + 1e-20
    w1 = s1 / denom
    w2 = s2 / denom
    probs_ref[...] = jnp.where(li == i1, w1, 0.0) + jnp.where(li == i2, w2, 0.0)


def _route_pallas(flat, rw_pad, bias_pad):
    return pl.pallas_call(
        _router_body,
        out_shape=jax.ShapeDtypeStruct((T, LANES), jnp.float32),
    )(flat, rw_pad, bias_pad)


BT = 512
BF = 512


def _ffn_body(x_ref, wg_ref, wu_ref, wd_ref, p_ref, o_ref):
    e = pl.program_id(1)
    f = pl.program_id(2)

    @pl.when((e == 0) & (f == 0))
    def _():
        o_ref[...] = jnp.zeros_like(o_ref)

    x = x_ref[...].astype(jnp.bfloat16)
    g = jnp.dot(x, wg_ref[0], preferred_element_type=jnp.float32)
    u = jnp.dot(x, wu_ref[0], preferred_element_type=jnp.float32)
    g = jnp.minimum(g, ALPHA)
    u = jnp.clip(u, -ALPHA, ALPHA)
    act = (g * jax.nn.sigmoid(g) * u).astype(jnp.bfloat16)
    y = jnp.dot(act, wd_ref[0], preferred_element_type=jnp.float32)
    o_ref[...] += y * p_ref[0]


def _ffn_dense(flat, Wg9, Wu9, Wd9, probs9):
    grid = (T // BT, E + 1, F // BF)
    return pl.pallas_call(
        _ffn_body,
        grid=grid,
        in_specs=[
            pl.BlockSpec((BT, D), lambda t, e, f: (t, 0)),
            pl.BlockSpec((1, D, BF), lambda t, e, f: (e, 0, f)),
            pl.BlockSpec((1, D, BF), lambda t, e, f: (e, 0, f)),
            pl.BlockSpec((1, BF, D), lambda t, e, f: (e, f, 0)),
            pl.BlockSpec((1, BT, 1), lambda t, e, f: (e, t, 0)),
        ],
        out_specs=pl.BlockSpec((BT, D), lambda t, e, f: (t, 0)),
        out_shape=jax.ShapeDtypeStruct((T, D), jnp.float32),
    )(flat, Wg9, Wu9, Wd9, probs9)


@jax.jit
def kernel(hidden, router_w, expert_bias, Wg, Wu, Wd, SWg, SWu, SWd):
    B, S, Dm = hidden.shape
    flat = hidden.reshape(T, D)

    rw_pad = jnp.zeros((D, LANES), jnp.float32).at[:, :E].set(router_w.T)
    bias_pad = jnp.zeros((1, LANES), jnp.float32).at[:, :E].set(
        expert_bias[None, :])
    probs = _route_pallas(flat, rw_pad, bias_pad)

    probs9 = jnp.concatenate(
        [probs[:, :E], jnp.ones((T, 1), jnp.float32)], axis=1)
    probs9 = probs9.T.reshape(E + 1, T, 1)

    Wg9 = jnp.concatenate([Wg, SWg[None]], axis=0).astype(jnp.bfloat16)
    Wu9 = jnp.concatenate([Wu, SWu[None]], axis=0).astype(jnp.bfloat16)
    Wd9 = jnp.concatenate([Wd, SWd[None]], axis=0).astype(jnp.bfloat16)

    out = _ffn_dense(flat, Wg9, Wu9, Wd9, probs9)
    return out.reshape(B, S, Dm)
